# Initial kernel scaffold; baseline (speedup 1.0000x reference)
import jax
import jax.numpy as jnp
from jax.experimental import pallas as pl


def kernel(x, table):
    raise NotImplementedError("write your pallas kernel here")


# speedup vs baseline: 2.8248x; 2.8248x over previous
import functools

import jax
import jax.numpy as jnp
from jax import lax
from jax.experimental import pallas as pl
from jax.experimental.pallas import tpu as pltpu
from jax.experimental.pallas import tpu_sc as plsc

B = 256
C = 3
H = W = 64
HW = H * W
D = 64
V = C * 256
NC, NS = 2, 16
NW = NC * NS
BPW = B // NW
CHUNK = 256
NCHUNK = HW // CHUNK
NPB = CHUNK // 16

_mesh = plsc.VectorSubcoreMesh(core_axis_name="c", subcore_axis_name="s")


@functools.partial(
    pl.kernel,
    mesh=_mesh,
    out_type=jax.ShapeDtypeStruct((B, D, HW), jnp.float32),
    scratch_types=[
        pltpu.VMEM((V * D,), jnp.float32),
        pltpu.VMEM((C, HW), jnp.int32),
        pltpu.VMEM((D, CHUNK), jnp.float32),
    ],
    compiler_params=pltpu.CompilerParams(needs_layout_passes=False),
)
def _bow_sc(x_hbm, table_hbm, out_hbm, table_v, x_v, o_v):
    wid = lax.axis_index("s") * NC + lax.axis_index("c")
    pltpu.sync_copy(table_hbm, table_v)

    def batch_body(i, carry):
        b = wid * BPW + i
        pltpu.sync_copy(x_hbm.at[b], x_v)

        def chunk_body(k, carry):
            def pb_body(pb, carry):
                off = k * CHUNK + pb * 16
                x0 = x_v[0, pl.ds(off, 16)]
                x1 = x_v[1, pl.ds(off, 16)]
                x2 = x_v[2, pl.ds(off, 16)]
                b0 = x0 * D
                b1 = x1 * D + 256 * D
                b2 = x2 * D + 512 * D
                for d in range(D):
                    acc = (plsc.load_gather(table_v, [b0 + d])
                           + plsc.load_gather(table_v, [b1 + d])
                           + plsc.load_gather(table_v, [b2 + d]))
                    o_v[d, pl.ds(pb * 16, 16)] = acc
                return carry

            lax.fori_loop(0, NPB, pb_body, 0)
            pltpu.sync_copy(o_v, out_hbm.at[b, :, pl.ds(k * CHUNK, CHUNK)])
            return carry

        lax.fori_loop(0, NCHUNK, chunk_body, 0)
        return carry

    lax.fori_loop(0, BPW, batch_body, 0)


def kernel(x, table):
    x3 = x.reshape(B, C, HW).astype(jnp.int32)
    out = _bow_sc(x3, table.reshape(-1))
    return out.reshape(B, D, H, W)


# speedup vs baseline: 9.1392x; 3.2353x over previous
import functools

import jax
import jax.numpy as jnp
from jax import lax
from jax.experimental import pallas as pl
from jax.experimental.pallas import tpu as pltpu
from jax.experimental.pallas import tpu_sc as plsc

B = 256
C = 3
H = W = 64
HW = H * W
D = 64
V = C * 256
NC, NS = 2, 16
NW = NC * NS
BPW = B // NW
DS = 65
CHUNK = 256
NCHUNK = HW // CHUNK
NPB = CHUNK // 16

_mesh = plsc.VectorSubcoreMesh(core_axis_name="c", subcore_axis_name="s")


@functools.partial(
    pl.kernel,
    mesh=_mesh,
    out_type=jax.ShapeDtypeStruct((B, D, HW), jnp.float32),
    scratch_types=[
        pltpu.VMEM((V * DS,), jnp.float32),
        pltpu.VMEM((C, HW), jnp.int32),
        pltpu.VMEM((D, CHUNK), jnp.float32),
    ],
    compiler_params=pltpu.CompilerParams(needs_layout_passes=False),
)
def _bow_sc(x_hbm, table_hbm, out_hbm, table_v, x_v, o_v):
    wid = lax.axis_index("s") * NC + lax.axis_index("c")
    pltpu.sync_copy(table_hbm, table_v)

    def batch_body(i, carry):
        b = wid * BPW + i
        pltpu.sync_copy(x_hbm.at[b], x_v)

        def chunk_body(k, carry):
            def pb_body(pb, carry):
                off = k * CHUNK + pb * 16
                x0 = x_v[0, pl.ds(off, 16)]
                x1 = x_v[1, pl.ds(off, 16)]
                x2 = x_v[2, pl.ds(off, 16)]
                b0 = x0 * DS
                b1 = x1 * DS + 256 * DS
                b2 = x2 * DS + 512 * DS
                for d in range(D):
                    acc = (plsc.load_gather(table_v, [b0 + d])
                           + plsc.load_gather(table_v, [b1 + d])
                           + plsc.load_gather(table_v, [b2 + d]))
                    o_v[d, pl.ds(pb * 16, 16)] = acc
                return carry

            lax.fori_loop(0, NPB, pb_body, 0)
            pltpu.sync_copy(o_v, out_hbm.at[b, :, pl.ds(k * CHUNK, CHUNK)])
            return carry

        lax.fori_loop(0, NCHUNK, chunk_body, 0)
        return carry

    lax.fori_loop(0, BPW, batch_body, 0)


def kernel(x, table):
    x3 = x.reshape(B, C, HW).astype(jnp.int32)
    tpad = jnp.pad(table, ((0, 0), (0, DS - D))).reshape(-1)
    out = _bow_sc(x3, tpad)
    return out.reshape(B, D, H, W)
